# Initial kernel scaffold; baseline (speedup 1.0000x reference)
import jax
import jax.numpy as jnp
from jax.experimental import pallas as pl


def kernel(x, item_emb, pos_emb):
    raise NotImplementedError("write your pallas kernel here")


# speedup vs baseline: 1.2532x; 1.2532x over previous
import functools

import jax
import jax.numpy as jnp
from jax import lax
from jax.experimental import pallas as pl
from jax.experimental.pallas import tpu as pltpu
from jax.experimental.pallas import tpu_sc as plsc


_LANES = 16
_NSPLIT = 2
_G = 4


def _make_kernel(B, S, D, V):
    info = plsc.get_sparse_core_info()
    NC, NS = info.num_cores, info.num_subcores
    NW = NC * NS
    assert B % NW == 0
    seq_per_w = B // NW
    assert seq_per_w % _G == 0
    n_chunks = seq_per_w // _G
    SS = S // _NSPLIT
    assert SS * _NSPLIT == S
    assert D % _LANES == 0
    HREG = D // _LANES

    mesh = plsc.VectorSubcoreMesh(core_axis_name="c", subcore_axis_name="s")

    @functools.partial(
        pl.kernel,
        mesh=mesh,
        compiler_params=pltpu.CompilerParams(use_tc_tiling_on_sc=False),
        out_type=jax.ShapeDtypeStruct((B, _NSPLIT, SS, D), jnp.float32),
        scratch_types=[
            pltpu.VMEM((_G, _NSPLIT, SS), jnp.int32),
            pltpu.VMEM((_G, _NSPLIT, SS, D), jnp.float32),
            pltpu.VMEM((_NSPLIT, SS, D), jnp.float32),
            pltpu.SemaphoreType.DMA,
        ],
    )
    def k(x_hbm, item_hbm, pos_hbm, out_hbm, idx_v, rows_v, pos_v, sem):
        wid = lax.axis_index("s") * NC + lax.axis_index("c")
        seq_base = wid * seq_per_w

        pltpu.sync_copy(pos_hbm, pos_v)

        def chunk_body(i, carry):
            b0 = seq_base + i * _G
            pltpu.sync_copy(x_hbm.at[pl.ds(b0, _G)], idx_v)

            copies = []
            for g in range(_G):
                for j in range(_NSPLIT):
                    copies.append(
                        pltpu.async_copy(
                            item_hbm.at[idx_v.at[g, j]], rows_v.at[g, j], sem
                        )
                    )
            for c in copies:
                c.wait()

            def add_body(r, c2):
                for j in range(_NSPLIT):
                    for h in range(HREG):
                        sl = pl.ds(h * _LANES, _LANES)
                        pv = pos_v[j, r, sl]
                        for g in range(_G):
                            rows_v[g, j, r, sl] = rows_v[g, j, r, sl] + pv
                return c2

            lax.fori_loop(0, SS, add_body, 0)

            pltpu.sync_copy(rows_v, out_hbm.at[pl.ds(b0, _G)])
            return carry

        lax.fori_loop(0, n_chunks, chunk_body, 0)

    return k


def kernel(x, item_emb, pos_emb):
    B, S = x.shape
    V, D = item_emb.shape
    x3 = x.astype(jnp.int32).reshape(B, _NSPLIT, S // _NSPLIT)
    pos3 = pos_emb[:S].reshape(_NSPLIT, S // _NSPLIT, D)
    out = _make_kernel(B, S, D, V)(x3, item_emb, pos3)
    return out.reshape(B, S, D)
